# baseline (speedup 1.0000x reference)
import functools

import jax
import jax.numpy as jnp
from jax.experimental import pallas as pl
from jax.experimental.pallas import tpu as pltpu
from jax.experimental.pallas import tpu_sc as plsc

VOCAB = 100000
D = 128
B = 128
CTX = 20

VBLK = 2048
NV = (VOCAB + VBLK - 1) // VBLK

GATHER_WINDOW = 128


def _sc_gather(table, idx_2d):
    mesh = plsc.VectorSubcoreMesh(core_axis_name="core", subcore_axis_name="subcore")

    @pl.kernel(
        out_type=jax.ShapeDtypeStruct((B, D), table.dtype),
        mesh=mesh,
    )
    def gather_kernel(table_hbm, idx_hbm, out_hbm):
        def body(i_vmem, o_vmem):
            pltpu.sync_copy(table_hbm.at[i_vmem.at[0]], o_vmem)

        pltpu.emit_pipeline(
            body,
            grid=(B // GATHER_WINDOW,),
            in_specs=[pl.BlockSpec((1, GATHER_WINDOW), lambda i: (0, i))],
            out_specs=[pl.BlockSpec((GATHER_WINDOW, D), lambda i: (i, 0))],
            core_axis_name=("core", "subcore"),
            dimension_semantics=(pltpu.PARALLEL,),
        )(idx_hbm, out_hbm)

    return gather_kernel(table, idx_2d)


def _tc_kernel(emb_ref, context_ref, w_ref, b_ref, table_ref, out_ref,
               logits_scr, lasth_scr, m_scr, s_scr):
    p = pl.program_id(0)
    j = pl.program_id(1)

    @pl.when((p == 0) & (j == 0))
    def _small_stage():
        ctx = jnp.mean(context_ref[...], axis=1)
        h = jnp.tanh(emb_ref[...] + ctx)
        pre = jax.lax.dot_general(
            h, w_ref[...], (((1,), (0,)), ((), ())),
            preferred_element_type=jnp.float32) + b_ref[...]
        lasth_scr[...] = jnp.tanh(pre)
        m_scr[...] = jnp.full((B, 1), -jnp.inf, jnp.float32)
        s_scr[...] = jnp.zeros((B, 1), jnp.float32)

    @pl.when(p == 0)
    def _phase0():
        lh = lasth_scr[...].astype(jnp.bfloat16)
        tb = table_ref[...].astype(jnp.bfloat16)
        logits = jax.lax.dot_general(
            lh, tb, (((1,), (1,)), ((), ())),
            preferred_element_type=jnp.float32)
        col = j * VBLK + jax.lax.broadcasted_iota(jnp.int32, (B, VBLK), 1)
        logits = jnp.where(col < VOCAB, logits, -jnp.inf)
        logits_scr[j] = logits.astype(jnp.bfloat16)
        bm = jnp.max(logits, axis=1, keepdims=True)
        m_old = m_scr[...]
        m_new = jnp.maximum(m_old, bm)
        bs = jnp.sum(jnp.exp(logits - m_new), axis=1, keepdims=True)
        s_scr[...] = s_scr[...] * jnp.exp(m_old - m_new) + bs
        m_scr[...] = m_new

    @pl.when(p == 1)
    def _phase1():
        lse = m_scr[...] + jnp.log(s_scr[...])
        out_ref[...] = logits_scr[j].astype(jnp.float32) - lse


@jax.jit
def kernel(prev_target, context, table, W, b):
    idx = prev_target[:, -1].astype(jnp.int32).reshape(1, B)
    emb = _sc_gather(table, idx)

    b2 = b.reshape(1, D)
    lp = pl.pallas_call(
        _tc_kernel,
        grid=(2, NV),
        in_specs=[
            pl.BlockSpec((B, D), lambda p, j: (0, 0)),
            pl.BlockSpec((B, CTX, D), lambda p, j: (0, 0, 0)),
            pl.BlockSpec((D, D), lambda p, j: (0, 0)),
            pl.BlockSpec((1, D), lambda p, j: (0, 0)),
            pl.BlockSpec((VBLK, D),
                         lambda p, j: (jax.lax.select(p == 0, j, NV - 1), 0)),
        ],
        out_specs=pl.BlockSpec((B, VBLK),
                               lambda p, j: (0, jax.lax.select(p == 0, 0, j))),
        out_shape=jax.ShapeDtypeStruct((B, VOCAB), jnp.float32),
        scratch_shapes=[
            pltpu.VMEM((NV, B, VBLK), jnp.bfloat16),
            pltpu.VMEM((B, D), jnp.float32),
            pltpu.VMEM((B, 1), jnp.float32),
            pltpu.VMEM((B, 1), jnp.float32),
        ],
    )(emb, context, W, b2, table)
    return lp
